# baseline (speedup 1.0000x reference)
import functools

import jax
import jax.numpy as jnp
from jax import lax
from jax.experimental import pallas as pl
from jax.experimental.pallas import tpu as pltpu
from jax.experimental.pallas import tpu_sc as plsc

MAXLEN = 200
EMBED = 32
BATCH = 4096

NC = 2
NS = 16
NW = NC * NS
LANES = 16

ROWS_W = BATCH // NW
CH_B = 4
NCHUNK = ROWS_W // CH_B
GROUPS = ((0, 120), (120, 80))
HALVES = EMBED // LANES

_mesh = plsc.VectorSubcoreMesh(core_axis_name="c", subcore_axis_name="s")


@functools.partial(
    pl.kernel,
    out_type=jax.ShapeDtypeStruct((BATCH, MAXLEN, EMBED), jnp.float32),
    mesh=_mesh,
    compiler_params=pltpu.CompilerParams(use_tc_tiling_on_sc=False),
    scratch_types=[
        pltpu.VMEM((CH_B, MAXLEN), jnp.int32),
        pltpu.VMEM((CH_B, MAXLEN, EMBED), jnp.float32),
        pltpu.VMEM((MAXLEN, EMBED), jnp.float32),
        pltpu.SemaphoreType.DMA,
    ],
)
def _embed_sc(x_hbm, tok_hbm, pos_hbm, out_hbm, idx_v, rows_v, pos_v, sem):
    wid = lax.axis_index("s") * NC + lax.axis_index("c")
    pltpu.sync_copy(pos_hbm, pos_v)

    def chunk_body(c, _):
        row0 = wid * ROWS_W + c * CH_B
        pltpu.sync_copy(x_hbm.at[pl.ds(row0, CH_B)], idx_v)
        copies = []
        for r in range(CH_B):
            for off, n in GROUPS:
                copies.append(
                    pltpu.async_copy(
                        tok_hbm.at[idx_v.at[r, pl.ds(off, n)]],
                        rows_v.at[r, pl.ds(off, n)],
                        sem,
                    )
                )
        for cp in copies:
            cp.wait()

        def pos_body(p, _):
            for r in range(CH_B):
                for h in range(HALVES):
                    sl = pl.ds(h * LANES, LANES)
                    rows_v[r, p, sl] = rows_v[r, p, sl] + pos_v[p, sl]
            return 0

        lax.fori_loop(0, MAXLEN, pos_body, 0, unroll=2)
        pltpu.sync_copy(rows_v, out_hbm.at[pl.ds(row0, CH_B)])
        return 0

    lax.fori_loop(0, NCHUNK, chunk_body, 0)


def kernel(x, token_table, pos_table):
    return _embed_sc(x.astype(jnp.int32), token_table, pos_table)
